# baseline (speedup 1.0000x reference)
import functools

import jax
import jax.numpy as jnp
from jax import lax
from jax.experimental import pallas as pl
from jax.experimental.pallas import tpu as pltpu
from jax.experimental.pallas import tpu_sc as plsc

_B = 16384
_D = 32
_G = 4
_L = 16
_NC = 2
_NS = 16
_NW = _NC * _NS
_BPW = _B // _NW
_CH = 128
_NCH = _BPW // _CH
_BLK = _CH // _L


def _build():
    mesh = plsc.VectorSubcoreMesh(core_axis_name="c", subcore_axis_name="s")

    @functools.partial(
        pl.kernel,
        mesh=mesh,
        out_type=jax.ShapeDtypeStruct((_B,), jnp.float32),
        scratch_types=[
            pltpu.VMEM((_NCH, _CH), jnp.int32),
            pltpu.VMEM((_NCH, _CH), jnp.int32),
            pltpu.VMEM((_NCH, _CH), jnp.int32),
            pltpu.VMEM((_NCH, _CH), jnp.int32),
            pltpu.VMEM((2, _CH, _G * _D), jnp.float32),
            pltpu.VMEM((2, _CH, _G * _D), jnp.float32),
            pltpu.VMEM((_BPW,), jnp.float32),
            pltpu.SemaphoreType.DMA,
        ],
        compiler_params=pltpu.CompilerParams(needs_layout_passes=False),
    )
    def k(uf_hbm, if_hbm, uidx_hbm, iidx_hbm, out_hbm,
          uidx_v, iidx_v, ugid_v, igid_v, ubuf, ibuf, out_v, sem):
        wid = lax.axis_index("s") * _NC + lax.axis_index("c")

        pltpu.sync_copy(uidx_hbm.at[pl.ds(wid * _NCH, _NCH)], uidx_v)
        pltpu.sync_copy(iidx_hbm.at[pl.ds(wid * _NCH, _NCH)], iidx_v)

        for j in range(_NCH):
            for blk in range(_BLK):
                s = pl.ds(blk * _L, _L)
                ugid_v[j, s] = lax.shift_right_logical(uidx_v[j, s], 2)
                igid_v[j, s] = lax.shift_right_logical(iidx_v[j, s], 2)

        def fire(j):
            p = j % 2
            return (pltpu.async_copy(uf_hbm.at[ugid_v.at[j]], ubuf.at[p], sem),
                    pltpu.async_copy(if_hbm.at[igid_v.at[j]], ibuf.at[p], sem))

        lane = lax.iota(jnp.int32, _L)
        inflight = fire(0)

        for j in range(_NCH):
            for c in inflight:
                c.wait()
            if j + 1 < _NCH:
                nxt = fire(j + 1)
            p = j % 2
            ub = ubuf.at[p]
            ib = ibuf.at[p]
            for blk in range(_BLK):
                s = pl.ds(blk * _L, _L)
                usub = lax.shift_left(uidx_v[j, s] & 3, 5)
                isub = lax.shift_left(iidx_v[j, s] & 3, 5)
                row = lane + blk * _L
                acc = jnp.zeros((_L,), jnp.float32)
                for d in range(_D):
                    acc = acc + \
                        plsc.load_gather(ub, [row, usub + d]) * \
                        plsc.load_gather(ib, [row, isub + d])
                out_v[pl.ds(j * _CH + blk * _L, _L)] = \
                    1.0 / (1.0 + jnp.exp(-acc))
            if j + 1 < _NCH:
                inflight = nxt

        pltpu.sync_copy(out_v, out_hbm.at[pl.ds(wid * _BPW, _BPW)])

    return k


_kernel_call = _build()


def kernel(X, user_factors, item_factors):
    Xi = X.astype(jnp.int32)
    uidx = Xi[:, 0].reshape(_NW * _NCH, _CH)
    iidx = Xi[:, 1].reshape(_NW * _NCH, _CH)
    uf = user_factors.reshape(-1, _G * _D)
    itf = item_factors.reshape(-1, _G * _D)
    out = _kernel_call(uf, itf, uidx, iidx)
    return out.reshape(_B, 1)


# speedup vs baseline: 3.3315x; 3.3315x over previous
import functools

import jax
import jax.numpy as jnp
from jax import lax
from jax.experimental import pallas as pl
from jax.experimental.pallas import tpu as pltpu
from jax.experimental.pallas import tpu_sc as plsc

_B = 16384
_D = 32
_L = 16
_NC = 2
_NS = 16
_NW = _NC * _NS
_BPW = _B // _NW
_RING = 8
_GRP = _BPW // _RING


def _build():
    mesh = plsc.VectorSubcoreMesh(core_axis_name="c", subcore_axis_name="s")

    @functools.partial(
        pl.kernel,
        mesh=mesh,
        out_type=jax.ShapeDtypeStruct((_B,), jnp.float32),
        scratch_types=[
            pltpu.VMEM((_BPW,), jnp.int32),
            pltpu.VMEM((_BPW,), jnp.int32),
            pltpu.VMEM((_RING, _D, 128), jnp.float32),
            pltpu.VMEM((_RING, _D, 128), jnp.float32),
            pltpu.VMEM((_D, _BPW), jnp.float32),
            pltpu.VMEM((_D, _BPW), jnp.float32),
            pltpu.VMEM((_BPW,), jnp.float32),
            pltpu.SemaphoreType.DMA,
        ],
        compiler_params=pltpu.CompilerParams(needs_layout_passes=False),
    )
    def k(uft_hbm, ift_hbm, uidx_hbm, iidx_hbm, out_hbm,
          uidx_v, iidx_v, ublk, iblk, uc, ic, out_v, sem):
        wid = lax.axis_index("s") * _NC + lax.axis_index("c")

        pltpu.sync_copy(uidx_hbm.at[pl.ds(wid * _BPW, _BPW)], uidx_v)
        pltpu.sync_copy(iidx_hbm.at[pl.ds(wid * _BPW, _BPW)], iidx_v)

        lane = lax.iota(jnp.int32, _L)
        lo = lane
        hi = lane + _L

        def group(g, carry):
            sgbase = g * _L
            uv16 = uidx_v[pl.ds(sgbase, _L)]
            iv16 = iidx_v[pl.ds(sgbase, _L)]
            for half in range(2):
                copies = []
                for s in range(_RING):
                    iu = uv16[half * _RING + s]
                    ii = iv16[half * _RING + s]
                    ub = lax.shift_right_logical(iu, 7) * 128
                    ib = lax.shift_right_logical(ii, 7) * 128
                    copies.append(pltpu.async_copy(
                        uft_hbm.at[:, pl.ds(pl.multiple_of(ub, 128), 128)],
                        ublk.at[s], sem))
                    copies.append(pltpu.async_copy(
                        ift_hbm.at[:, pl.ds(pl.multiple_of(ib, 128), 128)],
                        iblk.at[s], sem))
                for c in copies:
                    c.wait()
                for s in range(_RING):
                    r = sgbase + half * _RING + s
                    cu = jnp.full((_L,), uv16[half * _RING + s] & 127,
                                  jnp.int32)
                    ci = jnp.full((_L,), iv16[half * _RING + s] & 127,
                                  jnp.int32)
                    rv = jnp.full((_L,), r, jnp.int32)
                    plsc.store_scatter(
                        uc, [lo, rv], plsc.load_gather(ublk.at[s], [lo, cu]))
                    plsc.store_scatter(
                        uc, [hi, rv], plsc.load_gather(ublk.at[s], [hi, cu]))
                    plsc.store_scatter(
                        ic, [lo, rv], plsc.load_gather(iblk.at[s], [lo, ci]))
                    plsc.store_scatter(
                        ic, [hi, rv], plsc.load_gather(iblk.at[s], [hi, ci]))
            return carry

        lax.fori_loop(0, _BPW // _L, group, 0)

        def dot(blk, carry):
            s = pl.ds(blk * _L, _L)
            acc = uc[0, s] * ic[0, s]
            for d in range(1, _D):
                acc = acc + uc[d, s] * ic[d, s]
            out_v[s] = 1.0 / (1.0 + jnp.exp(-acc))
            return carry

        lax.fori_loop(0, _BPW // _L, dot, 0)

        pltpu.sync_copy(out_v, out_hbm.at[pl.ds(wid * _BPW, _BPW)])

    return k


_kernel_call = _build()


def kernel(X, user_factors, item_factors):
    Xi = X.astype(jnp.int32)
    uidx = Xi[:, 0]
    iidx = Xi[:, 1]
    out = _kernel_call(user_factors.T, item_factors.T, uidx, iidx)
    return out.reshape(_B, 1)


# speedup vs baseline: 3.4712x; 1.0420x over previous
import functools

import jax
import jax.numpy as jnp
from jax import lax
from jax.experimental import pallas as pl
from jax.experimental.pallas import tpu as pltpu
from jax.experimental.pallas import tpu_sc as plsc

_B = 16384
_D = 32
_L = 16
_NC = 2
_NS = 16
_N_USERS = 1000000
_UPW = 62464
_NCH = 61
_CW = 1024
_TAIL0 = 999424
_TAIL1 = 999936
_HCAP = 2048
_MCAP = 64
_SROWS = _B + 64


def _build_k1():
    mesh = plsc.VectorSubcoreMesh(core_axis_name="c", subcore_axis_name="s")

    @functools.partial(
        pl.kernel,
        mesh=mesh,
        out_type=jax.ShapeDtypeStruct((_NC, _SROWS, 128), jnp.float32),
        scratch_types=[
            pltpu.VMEM((_B,), jnp.int32),
            pltpu.VMEM((_HCAP + 16,), jnp.int32),
            pltpu.VMEM((_HCAP + 16,), jnp.int32),
            pltpu.VMEM((_D, _CW), jnp.float32),
            pltpu.VMEM((_D, _CW), jnp.float32),
            pltpu.VMEM((_D, 64), jnp.float32),
            pltpu.VMEM((_MCAP + 16,), jnp.int32),
            pltpu.VMEM((_MCAP + 16,), jnp.int32),
            pltpu.VMEM((2, 32), jnp.int32),
            pltpu.VMEM((_MCAP, 128), jnp.float32),
            pltpu.SemaphoreType.DMA,
            pltpu.SemaphoreType.DMA,
        ],
        compiler_params=pltpu.CompilerParams(needs_layout_passes=False),
    )
    def k1(uft, ift, uidx, iidx, upatch, ipatch, stage,
           idx_v, hu, hb, bufa, bufb, buft, mc, mb, rowid, srcb,
           sema, semb):
        sel = lax.axis_index("c")
        t = lax.axis_index("s")
        lane = lax.iota(jnp.int32, _L)
        lo = t * _UPW
        hi = jnp.where(t == _NS - 1, _N_USERS, lo + _UPW)
        garbage = _B + t * 2

        def sweep(table, idx_hbm, patch, sem_a, sem_b):
            pltpu.sync_copy(idx_hbm, idx_v)

            def filt(i, nh):
                iv = idx_v[pl.ds(i * _L, _L)]
                m = (iv >= lo) & (iv < hi)
                plsc.store_compressed(hu.at[pl.ds(nh, _L)], iv, mask=m)
                plsc.store_compressed(
                    hb.at[pl.ds(nh, _L)], lane + i * _L, mask=m)
                return nh + plsc.all_reduce_population_count(m)[0]

            nh = lax.fori_loop(0, _B // _L, filt, 0)
            nhv = nh // _L + 1
            hu[pl.ds(nh, _L)] = jnp.full((_L,), 0x7FFFFFFF, jnp.int32)

            def fetch(c, buf, sem):
                cb = lo + c * _CW
                pltpu.async_copy(
                    table.at[:, pl.ds(pl.multiple_of(cb, 128), _CW)],
                    buf, sem)

            def process(cb, span, buf):
                def match(v, nm):
                    huv = hu[pl.ds(v * _L, _L)]
                    hbv = hb[pl.ds(v * _L, _L)]
                    m = (huv >= cb) & (huv < cb + span)
                    plsc.store_compressed(
                        mc.at[pl.ds(nm, _L)], huv - cb, mask=m)
                    plsc.store_compressed(
                        mb.at[pl.ds(nm, _L)], hbv, mask=m)
                    return nm + plsc.all_reduce_population_count(m)[0]

                nm = lax.fori_loop(0, nhv, match, 0)

                def extract(g, carry):
                    s16 = pl.ds(g * _L, _L)
                    mcv = mc[s16]
                    mbv = mb[s16]
                    slot = lane + g * _L
                    act = slot < nm
                    plsc.store_scatter(
                        rowid,
                        [lax.shift_right_logical(slot, 5), slot & 31],
                        mbv, mask=act)
                    for d in range(_D):
                        val = plsc.load_gather(
                            buf, [jnp.full((_L,), d, jnp.int32), mcv],
                            mask=act)
                        plsc.store_scatter(
                            srcb, [slot, jnp.full((_L,), d, jnp.int32)],
                            val, mask=act)
                    return carry

                lax.fori_loop(0, (nm + _L - 1) // _L, extract, 0)

                pltpu.sync_copy(
                    srcb.at[pl.ds(0, 32)], stage.at[sel].at[rowid.at[0]])

                @pl.when(nm > 32)
                def _():
                    pltpu.sync_copy(
                        srcb.at[pl.ds(32, 32)],
                        stage.at[sel].at[rowid.at[1]])

                for j in range(2):
                    for g in range(2):
                        rowid[j, pl.ds(g * _L, _L)] = \
                            jnp.full((_L,), garbage, jnp.int32)

            for j in range(2):
                for g in range(2):
                    rowid[j, pl.ds(g * _L, _L)] = \
                        jnp.full((_L,), garbage, jnp.int32)

            fetch(0, bufa, sem_a)
            fetch(1, bufb, sem_b)

            def pair(p, carry):
                ca = p * 2
                pltpu.make_async_copy(
                    table.at[:, pl.ds(0, _CW)], bufa, sem_a).wait()
                process(lo + ca * _CW, _CW, bufa)
                fetch(ca + 2, bufa, sem_a)
                pltpu.make_async_copy(
                    table.at[:, pl.ds(0, _CW)], bufb, sem_b).wait()
                process(lo + (ca + 1) * _CW, _CW, bufb)

                @pl.when(ca + 3 < _NCH)
                def _():
                    fetch(ca + 3, bufb, sem_b)

                return carry

            lax.fori_loop(0, _NCH // 2, pair, 0)
            pltpu.make_async_copy(
                table.at[:, pl.ds(0, _CW)], bufa, sem_a).wait()
            process(lo + (_NCH - 1) * _CW, _CW, bufa)

            @pl.when(t == _NS - 1)
            def _():
                pltpu.async_copy(
                    table.at[:, pl.ds(pl.multiple_of(_TAIL0, 128), 512)],
                    bufa.at[:, pl.ds(0, 512)], sem_a)
                pltpu.sync_copy(patch, buft)
                pltpu.make_async_copy(
                    table.at[:, pl.ds(0, 512)],
                    bufa.at[:, pl.ds(0, 512)], sem_a).wait()
                process(_TAIL0, 512, bufa)
                process(_TAIL1, 64, buft)

        @pl.when(sel == 0)
        def _():
            sweep(uft, uidx, upatch, sema, semb)

        @pl.when(sel == 1)
        def _():
            sweep(ift, iidx, ipatch, sema, semb)

    return k1


def _build_k2():
    def body(stage_ref, out_ref):
        x = stage_ref[...]
        u = x[0, :, :_D]
        v = x[1, :, :_D]
        acc = jnp.sum(u * v, axis=-1)
        out_ref[...] = 1.0 / (1.0 + jnp.exp(-acc))

    blk = 512
    return pl.pallas_call(
        body,
        grid=(_B // blk,),
        in_specs=[pl.BlockSpec((_NC, blk, 128), lambda i: (0, i, 0))],
        out_specs=pl.BlockSpec((blk,), lambda i: (i,)),
        out_shape=jax.ShapeDtypeStruct((_B,), jnp.float32),
    )


_k1 = _build_k1()
_k2 = _build_k2()


def kernel(X, user_factors, item_factors):
    Xi = X.astype(jnp.int32)
    uidx = Xi[:, 0]
    iidx = Xi[:, 1]
    upatch = user_factors[_TAIL1:, :].T
    ipatch = item_factors[_TAIL1:, :].T
    stage = _k1(user_factors.T, item_factors.T, uidx, iidx, upatch, ipatch)
    out = _k2(stage)
    return out.reshape(_B, 1)
